# baseline (speedup 1.0000x reference)
import functools

import jax
import jax.numpy as jnp
from jax import lax
from jax.experimental import pallas as pl
from jax.experimental.pallas import tpu as pltpu
from jax.experimental.pallas import tpu_sc as plsc

N = 50000
E = 800000
D_IN = 4
H = 128
OUT = 8
G = 64

NC = 2
NS = 16
NW = NC * NS
L = 16

RANGE = 528
NRANGE = 96
NP = RANGE * NRANGE
PASSES = NRANGE // NW
BE = 128
BND_PAD = 112

_mesh = plsc.VectorSubcoreMesh(core_axis_name="c", subcore_axis_name="s")
_sc_params = pltpu.CompilerParams(needs_layout_passes=False)


def _wid():
    return lax.axis_index("s") * NC + lax.axis_index("c")


def _vextract(ref, p):
    b = (p // 8) * 8
    v = ref[pl.ds(b, 16)]
    lane = p - b
    return jnp.sum(jnp.where(jnp.arange(16, dtype=jnp.int32) == lane, v, 0))


STRIDE = 536
DEGW = 16


def _sel_store(colv, selbuf, base, sid, iota):
    for g in range(BE // 16):
        c16 = colv[pl.ds(g * 16, 16)]
        local = c16 - base
        valid = (local >= 0) & (local < RANGE)
        sel = jnp.where(valid, local, RANGE + (iota & 7))
        selbuf[pl.ds(g * 16, 16)] = sel + sid * STRIDE


@functools.partial(
    pl.kernel,
    mesh=_mesh,
    compiler_params=_sc_params,
    out_type=jax.ShapeDtypeStruct((NP, DEGW), jnp.float32),
    scratch_types=[
        pltpu.VMEM((BND_PAD,), jnp.int32),
        pltpu.VMEM((BE,), jnp.int32),
        pltpu.VMEM((BE,), jnp.int32),
        pltpu.VMEM((BE, DEGW), jnp.float32),
        pltpu.VMEM((BE, DEGW), jnp.float32),
        pltpu.VMEM_SHARED((NS * STRIDE, DEGW), jnp.float32),
    ],
)
def _deg_kernel(col_hbm, bnd_hbm, deg_hbm, bndv, colv, selbuf, onesb, zbuf, acc):
    wid = _wid()
    sid = lax.axis_index("s")
    pltpu.sync_copy(bnd_hbm, bndv)
    iota = jnp.arange(16, dtype=jnp.int32)

    def init_body(t, _):
        onesb[t, pl.ds(0, 16)] = jnp.ones((16,), jnp.float32)
        zbuf[t, pl.ds(0, 16)] = jnp.zeros((16,), jnp.float32)
        return 0

    lax.fori_loop(0, BE, init_body, 0)

    for p in range(PASSES):
        r = p * NW + wid
        base = r * RANGE
        e0 = _vextract(bndv, r)
        e1 = _vextract(bndv, r + 1)

        for q in range(4):
            pltpu.sync_copy(zbuf, acc.at[pl.ds(sid * STRIDE + q * BE, BE), :])
        pltpu.sync_copy(zbuf.at[pl.ds(0, STRIDE - 4 * BE), :],
                        acc.at[pl.ds(sid * STRIDE + 4 * BE, STRIDE - 4 * BE), :])

        c0 = (e0 // BE) * BE
        nchunk = (e1 - c0 + BE - 1) // BE

        def chunk_body(k, _):
            off = c0 + k * BE
            pltpu.sync_copy(col_hbm.at[pl.ds(off, BE)], colv)
            _sel_store(colv, selbuf, base, sid, iota)
            pltpu.sync_copy(onesb, acc.at[selbuf], add=True)
            return 0

        lax.fori_loop(0, nchunk, chunk_body, 0)
        pltpu.sync_copy(acc.at[pl.ds(sid * STRIDE, RANGE), :],
                        deg_hbm.at[pl.ds(base, RANGE), :])


SUP = 16
EBIG = SUP * BE


@functools.partial(
    pl.kernel,
    mesh=_mesh,
    compiler_params=_sc_params,
    out_type=jax.ShapeDtypeStruct((NP, H), jnp.float32),
    scratch_types=[
        pltpu.VMEM((BND_PAD,), jnp.int32),
        pltpu.VMEM((EBIG,), jnp.int32),
        pltpu.VMEM((EBIG,), jnp.int32),
        pltpu.VMEM((BE,), jnp.int32),
        pltpu.VMEM((BE,), jnp.int32),
        pltpu.VMEM((BE, H), jnp.float32),
        pltpu.VMEM((BE, H), jnp.float32),
        pltpu.VMEM((BE, H), jnp.float32),
        pltpu.VMEM_SHARED((NS * STRIDE, H), jnp.float32),
        pltpu.SemaphoreType.DMA,
        pltpu.SemaphoreType.DMA,
    ],
)
def _agg_kernel(hs_hbm, row_hbm, col_hbm, bnd_hbm, agg_hbm,
                bndv, rowbig, colbig, selb0, selb1, rbuf0, rbuf1, zbuf,
                acc, sem0, sem1):
    wid = _wid()
    sid = lax.axis_index("s")
    pltpu.sync_copy(bnd_hbm, bndv)
    iota = jnp.arange(16, dtype=jnp.int32)
    rbufs = (rbuf0, rbuf1)
    selbs = (selb0, selb1)
    sems = (sem0, sem1)

    def init_body(t, _):
        for j in range(H // 16):
            zbuf[t, pl.ds(j * 16, 16)] = jnp.zeros((16,), jnp.float32)
        return 0

    lax.fori_loop(0, BE, init_body, 0)

    for p in range(PASSES):
        r = p * NW + wid
        base = r * RANGE
        e0 = _vextract(bndv, r)
        e1 = _vextract(bndv, r + 1)

        for q in range(4):
            pltpu.sync_copy(zbuf, acc.at[pl.ds(sid * STRIDE + q * BE, BE), :])
        pltpu.sync_copy(zbuf.at[pl.ds(0, STRIDE - 4 * BE), :],
                        acc.at[pl.ds(sid * STRIDE + 4 * BE, STRIDE - 4 * BE), :])

        c0 = (e0 // EBIG) * EBIG
        nchunk = (e1 - c0 + BE - 1) // BE
        nsuper = (e1 - c0 + EBIG - 1) // EBIG

        def super_body(ss, _):
            soff = c0 + ss * EBIG
            pltpu.sync_copy(row_hbm.at[pl.ds(soff, EBIG)], rowbig)
            pltpu.sync_copy(col_hbm.at[pl.ds(soff, EBIG)], colbig)
            kbase = ss * SUP

            @pl.when(kbase < nchunk)
            def _():
                pltpu.async_copy(hs_hbm.at[rowbig.at[pl.ds(0, BE)]],
                                 rbufs[0], sems[0])

            for j in range(SUP):
                b = j & 1

                @pl.when(kbase + j < nchunk)
                def _():
                    if j + 1 < SUP:
                        @pl.when(kbase + j + 1 < nchunk)
                        def _():
                            pltpu.async_copy(
                                hs_hbm.at[rowbig.at[pl.ds((j + 1) * BE, BE)]],
                                rbufs[1 - b], sems[1 - b])
                    for g in range(BE // 16):
                        c16 = colbig[pl.ds(j * BE + g * 16, 16)]
                        local = c16 - base
                        valid = (local >= 0) & (local < RANGE)
                        sel = jnp.where(valid, local, RANGE + (iota & 7))
                        selbs[b][pl.ds(g * 16, 16)] = sel + sid * STRIDE
                    pltpu.make_async_copy(hs_hbm.at[rowbig.at[pl.ds(j * BE, BE)]],
                                          rbufs[b], sems[b]).wait()
                    pltpu.sync_copy(rbufs[b], acc.at[selbs[b]], add=True)
            return 0

        lax.fori_loop(0, nsuper, super_body, 0)
        pltpu.sync_copy(acc.at[pl.ds(sid * STRIDE, RANGE), :],
                        agg_hbm.at[pl.ds(base, RANGE), :])


@functools.partial(
    pl.kernel,
    mesh=_mesh,
    compiler_params=_sc_params,
    out_type=[
        jax.ShapeDtypeStruct((G * H,), jnp.float32),
        jax.ShapeDtypeStruct((G * H,), jnp.float32),
    ],
    scratch_types=[
        pltpu.VMEM((N,), jnp.int32),
        pltpu.VMEM((64, H), jnp.float32),
        pltpu.VMEM((H,), jnp.float32),
        pltpu.VMEM((H,), jnp.float32),
    ],
)
def _pool_kernel(hid_hbm, batch_hbm, gmp_hbm, gap_hbm,
                 batchv, hbuf, macc, sacc):
    wid = _wid()
    pltpu.sync_copy(batch_hbm, batchv)
    g0 = wid * 2

    def cnt_body(t, carry):
        c0, c1, c2 = carry
        b16 = batchv[pl.ds(t * 16, 16)]
        one = jnp.ones((16,), jnp.int32)
        zero = jnp.zeros((16,), jnp.int32)
        c0 = c0 + jnp.where(b16 < g0, one, zero)
        c1 = c1 + jnp.where(b16 < g0 + 1, one, zero)
        c2 = c2 + jnp.where(b16 < g0 + 2, one, zero)
        return (c0, c1, c2)

    z = jnp.zeros((16,), jnp.int32)
    c0, c1, c2 = lax.fori_loop(0, N // 16, cnt_body, (z, z, z))
    s0 = jnp.sum(c0)
    s1 = jnp.sum(c1)
    s2 = jnp.sum(c2)

    for which in range(2):
        g = g0 + which
        s = jnp.where(which == 0, s0, s1)
        e = jnp.where(which == 0, s1, s2)
        for k in range(H // 16):
            macc[pl.ds(k * 16, 16)] = jnp.full((16,), -jnp.inf, jnp.float32)
            sacc[pl.ds(k * 16, 16)] = jnp.zeros((16,), jnp.float32)

        sa = (s // 8) * 8
        nchunk = (e - sa + 63) // 64

        def chunk_body(kk, _):
            rs = sa + kk * 64
            pltpu.sync_copy(hid_hbm.at[pl.ds(rs, 64), :], hbuf)
            lo = jnp.maximum(s - rs, 0)
            hi = jnp.minimum(64, e - rs)

            def row_body(j, _):
                for k in range(H // 16):
                    v = hbuf[j, pl.ds(k * 16, 16)]
                    m = macc[pl.ds(k * 16, 16)]
                    macc[pl.ds(k * 16, 16)] = jnp.maximum(m, v)
                    sv = sacc[pl.ds(k * 16, 16)]
                    sacc[pl.ds(k * 16, 16)] = sv + v
                return 0

            lax.fori_loop(lo, hi, row_body, 0)
            return 0

        lax.fori_loop(0, nchunk, chunk_body, 0)
        cf = (e - s).astype(jnp.float32)
        for k in range(H // 16):
            sv = sacc[pl.ds(k * 16, 16)]
            sacc[pl.ds(k * 16, 16)] = sv / cf
        pltpu.sync_copy(macc, gmp_hbm.at[pl.ds(g * H, H)])
        pltpu.sync_copy(sacc, gap_hbm.at[pl.ds(g * H, H)])


def _tc_layer0(x_pad, deg, W0, b0):
    def body(x_ref, deg_ref, w_ref, b_ref, hs_ref, dinv_ref):
        z = jnp.dot(x_ref[...], w_ref[...],
                    preferred_element_type=jnp.float32) + b_ref[...]
        dinv = lax.rsqrt(deg_ref[...][:, 0:1] + 1.0)
        hs_ref[...] = dinv * z
        dinv_ref[...] = dinv

    grid = (NP // 512,)
    return pl.pallas_call(
        body,
        grid=grid,
        in_specs=[
            pl.BlockSpec((512, D_IN), lambda i: (i, 0)),
            pl.BlockSpec((512, DEGW), lambda i: (i, 0)),
            pl.BlockSpec((D_IN, H), lambda i: (0, 0)),
            pl.BlockSpec((1, H), lambda i: (0, 0)),
        ],
        out_specs=[
            pl.BlockSpec((512, H), lambda i: (i, 0)),
            pl.BlockSpec((512, 1), lambda i: (i, 0)),
        ],
        out_shape=[
            jax.ShapeDtypeStruct((NP, H), jnp.float32),
            jax.ShapeDtypeStruct((NP, 1), jnp.float32),
        ],
    )(x_pad, deg, W0, b0)


def _tc_layer(agg, hs_prev, dinv, W, b):
    def body(a_ref, h_ref, d_ref, w_ref, b_ref, o_ref):
        t = jnp.tanh(d_ref[...] * (a_ref[...] + h_ref[...]))
        z = jnp.dot(t, w_ref[...], preferred_element_type=jnp.float32) + b_ref[...]
        o_ref[...] = d_ref[...] * z

    grid = (NP // 512,)
    return pl.pallas_call(
        body,
        grid=grid,
        in_specs=[
            pl.BlockSpec((512, H), lambda i: (i, 0)),
            pl.BlockSpec((512, H), lambda i: (i, 0)),
            pl.BlockSpec((512, 1), lambda i: (i, 0)),
            pl.BlockSpec((H, H), lambda i: (0, 0)),
            pl.BlockSpec((1, H), lambda i: (0, 0)),
        ],
        out_specs=pl.BlockSpec((512, H), lambda i: (i, 0)),
        out_shape=jax.ShapeDtypeStruct((NP, H), jnp.float32),
    )(agg, hs_prev, dinv, W, b)


def _tc_final(agg, hs_prev, dinv, topo_pad, Wf, bf):
    def body(a_ref, h_ref, d_ref, t_ref, w_ref, b_ref, o_ref):
        t = jnp.tanh(d_ref[...] * (a_ref[...] + h_ref[...]))
        comb = t_ref[...] * t
        o_ref[...] = jnp.dot(comb, w_ref[...],
                             preferred_element_type=jnp.float32) + b_ref[...]

    grid = (NP // 512,)
    return pl.pallas_call(
        body,
        grid=grid,
        in_specs=[
            pl.BlockSpec((512, H), lambda i: (i, 0)),
            pl.BlockSpec((512, H), lambda i: (i, 0)),
            pl.BlockSpec((512, 1), lambda i: (i, 0)),
            pl.BlockSpec((512, H), lambda i: (i, 0)),
            pl.BlockSpec((H, H), lambda i: (0, 0)),
            pl.BlockSpec((1, H), lambda i: (0, 0)),
        ],
        out_specs=pl.BlockSpec((512, H), lambda i: (i, 0)),
        out_shape=jax.ShapeDtypeStruct((NP, H), jnp.float32),
    )(agg, hs_prev, dinv, topo_pad, Wf, bf)


def _tc_head(pooled, Wo, bo):
    def body(p_ref, w_ref, b_ref, o_ref):
        o_ref[...] = jnp.dot(p_ref[...], w_ref[...],
                             preferred_element_type=jnp.float32) + b_ref[...]

    return pl.pallas_call(
        body,
        in_specs=[
            pl.BlockSpec((G, 2 * H), lambda: (0, 0)),
            pl.BlockSpec((2 * H, OUT), lambda: (0, 0)),
            pl.BlockSpec((1, OUT), lambda: (0, 0)),
        ],
        out_specs=pl.BlockSpec((G, OUT), lambda: (0, 0)),
        out_shape=jax.ShapeDtypeStruct((G, OUT), jnp.float32),
    )(pooled, Wo, bo)


def kernel(x, edge_index, batch_index, topo_hidden,
           W0, b0, W1, b1, W2, b2, W3, b3, Wf, bf, Wo, bo):
    row = edge_index[0]
    col = edge_index[1]
    col_s, row_s = lax.sort((col, row), num_keys=1)
    epad = ((E + EBIG - 1) // EBIG) * EBIG - E
    row_s = jnp.concatenate([row_s, jnp.zeros((epad,), jnp.int32)])
    col_sp = jnp.concatenate([col_s, jnp.full((epad,), N, jnp.int32)])
    cuts = jnp.arange(NRANGE + 1, dtype=jnp.int32) * RANGE
    bnd = jnp.searchsorted(col_s, cuts).astype(jnp.int32)
    bnd = jnp.concatenate(
        [bnd, jnp.full((BND_PAD - NRANGE - 1,), E, jnp.int32)])

    x_pad = jnp.pad(x, ((0, NP - N), (0, 0)))
    topo_pad = jnp.pad(topo_hidden, ((0, NP - N), (0, 0)))

    deg = _deg_kernel(col_sp, bnd)

    hs, dinv = _tc_layer0(x_pad, deg, W0, b0.reshape(1, H))
    for W, b in ((W1, b1), (W2, b2), (W3, b3)):
        agg = _agg_kernel(hs, row_s, col_sp, bnd)
        hs = _tc_layer(agg, hs, dinv, W, b.reshape(1, H))
    agg = _agg_kernel(hs, row_s, col_sp, bnd)
    hidden = _tc_final(agg, hs, dinv, topo_pad, Wf, bf.reshape(1, H))

    gmp, gap = _pool_kernel(hidden, batch_index)
    pooled = jnp.concatenate([gmp.reshape(G, H), gap.reshape(G, H)], axis=1)
    out = _tc_head(pooled, Wo, bo.reshape(1, OUT))
    return (out, pooled)


# speedup vs baseline: 1.3833x; 1.3833x over previous
import functools

import jax
import jax.numpy as jnp
from jax import lax
from jax.experimental import pallas as pl
from jax.experimental.pallas import tpu as pltpu
from jax.experimental.pallas import tpu_sc as plsc

N = 50000
E = 800000
D_IN = 4
H = 128
OUT = 8
G = 64

NC = 2
NS = 16
NW = NC * NS
L = 16

RANGE = 528
NRANGE = 96
NP = RANGE * NRANGE
PASSES = NRANGE // NW
BE = 128
BND_PAD = 112

_mesh = plsc.VectorSubcoreMesh(core_axis_name="c", subcore_axis_name="s")
_sc_params = pltpu.CompilerParams(needs_layout_passes=False)


def _wid():
    return lax.axis_index("s") * NC + lax.axis_index("c")


def _vextract(ref, p):
    b = (p // 8) * 8
    v = ref[pl.ds(b, 16)]
    lane = p - b
    return jnp.sum(jnp.where(jnp.arange(16, dtype=jnp.int32) == lane, v, 0))


STRIDE = 536
DEGW = 16


def _sel_store(colv, selbuf, base, sid, iota):
    for g in range(BE // 16):
        c16 = colv[pl.ds(g * 16, 16)]
        local = c16 - base
        valid = (local >= 0) & (local < RANGE)
        sel = jnp.where(valid, local, RANGE + (iota & 7))
        selbuf[pl.ds(g * 16, 16)] = sel + sid * STRIDE


@functools.partial(
    pl.kernel,
    mesh=_mesh,
    compiler_params=_sc_params,
    out_type=jax.ShapeDtypeStruct((NP, DEGW), jnp.float32),
    scratch_types=[
        pltpu.VMEM((BND_PAD,), jnp.int32),
        pltpu.VMEM((BE,), jnp.int32),
        pltpu.VMEM((BE,), jnp.int32),
        pltpu.VMEM((BE, DEGW), jnp.float32),
        pltpu.VMEM((BE, DEGW), jnp.float32),
        pltpu.VMEM_SHARED((NS * STRIDE, DEGW), jnp.float32),
    ],
)
def _deg_kernel(col_hbm, bnd_hbm, deg_hbm, bndv, colv, selbuf, onesb, zbuf, acc):
    wid = _wid()
    sid = lax.axis_index("s")
    pltpu.sync_copy(bnd_hbm, bndv)
    iota = jnp.arange(16, dtype=jnp.int32)

    def init_body(t, _):
        onesb[t, pl.ds(0, 16)] = jnp.ones((16,), jnp.float32)
        zbuf[t, pl.ds(0, 16)] = jnp.zeros((16,), jnp.float32)
        return 0

    lax.fori_loop(0, BE, init_body, 0)

    for p in range(PASSES):
        r = p * NW + wid
        base = r * RANGE
        e0 = _vextract(bndv, r)
        e1 = _vextract(bndv, r + 1)

        for q in range(4):
            pltpu.sync_copy(zbuf, acc.at[pl.ds(sid * STRIDE + q * BE, BE), :])
        pltpu.sync_copy(zbuf.at[pl.ds(0, STRIDE - 4 * BE), :],
                        acc.at[pl.ds(sid * STRIDE + 4 * BE, STRIDE - 4 * BE), :])

        c0 = (e0 // BE) * BE
        nchunk = (e1 - c0 + BE - 1) // BE

        def chunk_body(k, _):
            off = c0 + k * BE
            pltpu.sync_copy(col_hbm.at[pl.ds(off, BE)], colv)
            _sel_store(colv, selbuf, base, sid, iota)
            pltpu.sync_copy(onesb, acc.at[selbuf], add=True)
            return 0

        lax.fori_loop(0, nchunk, chunk_body, 0)
        pltpu.sync_copy(acc.at[pl.ds(sid * STRIDE, RANGE), :],
                        deg_hbm.at[pl.ds(base, RANGE), :])


SUP = 16
EBIG = SUP * BE


@functools.partial(
    pl.kernel,
    mesh=_mesh,
    compiler_params=_sc_params,
    out_type=jax.ShapeDtypeStruct((NP, H), jnp.float32),
    scratch_types=[
        pltpu.VMEM((BND_PAD,), jnp.int32),
        pltpu.VMEM((EBIG,), jnp.int32),
        pltpu.VMEM((EBIG,), jnp.int32),
        pltpu.VMEM((BE,), jnp.int32),
        pltpu.VMEM((BE,), jnp.int32),
        pltpu.VMEM((BE, H), jnp.float32),
        pltpu.VMEM((BE, H), jnp.float32),
        pltpu.VMEM((BE, H), jnp.float32),
        pltpu.VMEM_SHARED((NS * STRIDE, H), jnp.float32),
        pltpu.SemaphoreType.DMA,
        pltpu.SemaphoreType.DMA,
    ],
)
def _agg_kernel(hs_hbm, row_hbm, col_hbm, bnd_hbm, agg_hbm,
                bndv, rowbig, colbig, selb0, selb1, rbuf0, rbuf1, zbuf,
                acc, sem0, sem1):
    wid = _wid()
    sid = lax.axis_index("s")
    pltpu.sync_copy(bnd_hbm, bndv)
    iota = jnp.arange(16, dtype=jnp.int32)
    rbufs = (rbuf0, rbuf1)
    selbs = (selb0, selb1)
    sems = (sem0, sem1)

    def init_body(t, _):
        for j in range(H // 16):
            zbuf[t, pl.ds(j * 16, 16)] = jnp.zeros((16,), jnp.float32)
        return 0

    lax.fori_loop(0, BE, init_body, 0)

    for p in range(PASSES):
        r = p * NW + wid
        base = r * RANGE
        e0 = _vextract(bndv, r)
        e1 = _vextract(bndv, r + 1)

        for q in range(4):
            pltpu.sync_copy(zbuf, acc.at[pl.ds(sid * STRIDE + q * BE, BE), :])
        pltpu.sync_copy(zbuf.at[pl.ds(0, STRIDE - 4 * BE), :],
                        acc.at[pl.ds(sid * STRIDE + 4 * BE, STRIDE - 4 * BE), :])

        c0 = (e0 // EBIG) * EBIG
        nchunk = (e1 - c0 + BE - 1) // BE
        nsuper = (e1 - c0 + EBIG - 1) // EBIG

        def super_body(ss, _):
            soff = c0 + ss * EBIG
            pltpu.sync_copy(row_hbm.at[pl.ds(soff, EBIG)], rowbig)
            pltpu.sync_copy(col_hbm.at[pl.ds(soff, EBIG)], colbig)
            kbase = ss * SUP

            @pl.when(kbase < nchunk)
            def _():
                pltpu.async_copy(hs_hbm.at[rowbig.at[pl.ds(0, BE)]],
                                 rbufs[0], sems[0])

            for j in range(SUP):
                b = j & 1

                @pl.when(kbase + j < nchunk)
                def _():
                    if j + 1 < SUP:
                        @pl.when(kbase + j + 1 < nchunk)
                        def _():
                            pltpu.async_copy(
                                hs_hbm.at[rowbig.at[pl.ds((j + 1) * BE, BE)]],
                                rbufs[1 - b], sems[1 - b])
                    for g in range(BE // 16):
                        c16 = colbig[pl.ds(j * BE + g * 16, 16)]
                        local = c16 - base
                        valid = (local >= 0) & (local < RANGE)
                        sel = jnp.where(valid, local, RANGE + (iota & 7))
                        selbs[b][pl.ds(g * 16, 16)] = sel + sid * STRIDE
                    pltpu.make_async_copy(hs_hbm.at[rowbig.at[pl.ds(j * BE, BE)]],
                                          rbufs[b], sems[b]).wait()
                    pltpu.sync_copy(rbufs[b], acc.at[selbs[b]], add=True)
            return 0

        lax.fori_loop(0, nsuper, super_body, 0)
        pltpu.sync_copy(acc.at[pl.ds(sid * STRIDE, RANGE), :],
                        agg_hbm.at[pl.ds(base, RANGE), :])


@functools.partial(
    pl.kernel,
    mesh=_mesh,
    compiler_params=_sc_params,
    out_type=[
        jax.ShapeDtypeStruct((G * H,), jnp.float32),
        jax.ShapeDtypeStruct((G * H,), jnp.float32),
    ],
    scratch_types=[
        pltpu.VMEM((N,), jnp.int32),
        pltpu.VMEM((64, H), jnp.float32),
        pltpu.VMEM((H,), jnp.float32),
        pltpu.VMEM((H,), jnp.float32),
    ],
)
def _pool_kernel(hid_hbm, batch_hbm, gmp_hbm, gap_hbm,
                 batchv, hbuf, macc, sacc):
    wid = _wid()
    pltpu.sync_copy(batch_hbm, batchv)
    g0 = wid * 2

    def cnt_body(t, carry):
        c0, c1, c2 = carry
        b16 = batchv[pl.ds(t * 16, 16)]
        one = jnp.ones((16,), jnp.int32)
        zero = jnp.zeros((16,), jnp.int32)
        c0 = c0 + jnp.where(b16 < g0, one, zero)
        c1 = c1 + jnp.where(b16 < g0 + 1, one, zero)
        c2 = c2 + jnp.where(b16 < g0 + 2, one, zero)
        return (c0, c1, c2)

    z = jnp.zeros((16,), jnp.int32)
    c0, c1, c2 = lax.fori_loop(0, N // 16, cnt_body, (z, z, z))
    s0 = jnp.sum(c0)
    s1 = jnp.sum(c1)
    s2 = jnp.sum(c2)

    for which in range(2):
        g = g0 + which
        s = jnp.where(which == 0, s0, s1)
        e = jnp.where(which == 0, s1, s2)
        for k in range(H // 16):
            macc[pl.ds(k * 16, 16)] = jnp.full((16,), -jnp.inf, jnp.float32)
            sacc[pl.ds(k * 16, 16)] = jnp.zeros((16,), jnp.float32)

        sa = (s // 8) * 8
        nchunk = (e - sa + 63) // 64

        def chunk_body(kk, _):
            rs = sa + kk * 64
            pltpu.sync_copy(hid_hbm.at[pl.ds(rs, 64), :], hbuf)
            lo = jnp.maximum(s - rs, 0)
            hi = jnp.minimum(64, e - rs)

            def row_body(j, _):
                for k in range(H // 16):
                    v = hbuf[j, pl.ds(k * 16, 16)]
                    m = macc[pl.ds(k * 16, 16)]
                    macc[pl.ds(k * 16, 16)] = jnp.maximum(m, v)
                    sv = sacc[pl.ds(k * 16, 16)]
                    sacc[pl.ds(k * 16, 16)] = sv + v
                return 0

            lax.fori_loop(lo, hi, row_body, 0)
            return 0

        lax.fori_loop(0, nchunk, chunk_body, 0)
        cf = (e - s).astype(jnp.float32)
        for k in range(H // 16):
            sv = sacc[pl.ds(k * 16, 16)]
            sacc[pl.ds(k * 16, 16)] = sv / cf
        pltpu.sync_copy(macc, gmp_hbm.at[pl.ds(g * H, H)])
        pltpu.sync_copy(sacc, gap_hbm.at[pl.ds(g * H, H)])


def _tc_layer0(x_pad, deg, W0, b0):
    def body(x_ref, deg_ref, w_ref, b_ref, hs_ref, dinv_ref):
        z = jnp.dot(x_ref[...], w_ref[...],
                    preferred_element_type=jnp.float32) + b_ref[...]
        dinv = lax.rsqrt(deg_ref[...][:, 0:1] + 1.0)
        hs_ref[...] = dinv * z
        dinv_ref[...] = dinv

    grid = (NP // 512,)
    return pl.pallas_call(
        body,
        grid=grid,
        in_specs=[
            pl.BlockSpec((512, D_IN), lambda i: (i, 0)),
            pl.BlockSpec((512, DEGW), lambda i: (i, 0)),
            pl.BlockSpec((D_IN, H), lambda i: (0, 0)),
            pl.BlockSpec((1, H), lambda i: (0, 0)),
        ],
        out_specs=[
            pl.BlockSpec((512, H), lambda i: (i, 0)),
            pl.BlockSpec((512, 1), lambda i: (i, 0)),
        ],
        out_shape=[
            jax.ShapeDtypeStruct((NP, H), jnp.float32),
            jax.ShapeDtypeStruct((NP, 1), jnp.float32),
        ],
    )(x_pad, deg, W0, b0)


def _tc_layer(agg, hs_prev, dinv, W, b):
    def body(a_ref, h_ref, d_ref, w_ref, b_ref, o_ref):
        t = jnp.tanh(d_ref[...] * (a_ref[...] + h_ref[...]))
        z = jnp.dot(t, w_ref[...], preferred_element_type=jnp.float32) + b_ref[...]
        o_ref[...] = d_ref[...] * z

    grid = (NP // 512,)
    return pl.pallas_call(
        body,
        grid=grid,
        in_specs=[
            pl.BlockSpec((512, H), lambda i: (i, 0)),
            pl.BlockSpec((512, H), lambda i: (i, 0)),
            pl.BlockSpec((512, 1), lambda i: (i, 0)),
            pl.BlockSpec((H, H), lambda i: (0, 0)),
            pl.BlockSpec((1, H), lambda i: (0, 0)),
        ],
        out_specs=pl.BlockSpec((512, H), lambda i: (i, 0)),
        out_shape=jax.ShapeDtypeStruct((NP, H), jnp.float32),
    )(agg, hs_prev, dinv, W, b)


def _tc_final(agg, hs_prev, dinv, topo_pad, Wf, bf):
    def body(a_ref, h_ref, d_ref, t_ref, w_ref, b_ref, o_ref):
        t = jnp.tanh(d_ref[...] * (a_ref[...] + h_ref[...]))
        comb = t_ref[...] * t
        o_ref[...] = jnp.dot(comb, w_ref[...],
                             preferred_element_type=jnp.float32) + b_ref[...]

    grid = (NP // 512,)
    return pl.pallas_call(
        body,
        grid=grid,
        in_specs=[
            pl.BlockSpec((512, H), lambda i: (i, 0)),
            pl.BlockSpec((512, H), lambda i: (i, 0)),
            pl.BlockSpec((512, 1), lambda i: (i, 0)),
            pl.BlockSpec((512, H), lambda i: (i, 0)),
            pl.BlockSpec((H, H), lambda i: (0, 0)),
            pl.BlockSpec((1, H), lambda i: (0, 0)),
        ],
        out_specs=pl.BlockSpec((512, H), lambda i: (i, 0)),
        out_shape=jax.ShapeDtypeStruct((NP, H), jnp.float32),
    )(agg, hs_prev, dinv, topo_pad, Wf, bf)


def _tc_head(pooled, Wo, bo):
    def body(p_ref, w_ref, b_ref, o_ref):
        o_ref[...] = jnp.dot(p_ref[...], w_ref[...],
                             preferred_element_type=jnp.float32) + b_ref[...]

    return pl.pallas_call(
        body,
        in_specs=[
            pl.BlockSpec((G, 2 * H), lambda: (0, 0)),
            pl.BlockSpec((2 * H, OUT), lambda: (0, 0)),
            pl.BlockSpec((1, OUT), lambda: (0, 0)),
        ],
        out_specs=pl.BlockSpec((G, OUT), lambda: (0, 0)),
        out_shape=jax.ShapeDtypeStruct((G, OUT), jnp.float32),
    )(pooled, Wo, bo)


def kernel(x, edge_index, batch_index, topo_hidden,
           W0, b0, W1, b1, W2, b2, W3, b3, Wf, bf, Wo, bo):
    row = edge_index[0]
    col = edge_index[1]
    key = (col.astype(jnp.uint32) << 16) | row.astype(jnp.uint32)
    key_s = lax.sort(key)
    col_s = (key_s >> 16).astype(jnp.int32)
    row_s = (key_s & 0xFFFF).astype(jnp.int32)
    epad = ((E + EBIG - 1) // EBIG) * EBIG - E
    row_s = jnp.concatenate([row_s, jnp.zeros((epad,), jnp.int32)])
    col_sp = jnp.concatenate([col_s, jnp.full((epad,), N, jnp.int32)])
    cuts = jnp.arange(NRANGE + 1, dtype=jnp.int32) * RANGE
    bnd = jnp.searchsorted(col_s, cuts).astype(jnp.int32)
    bnd = jnp.concatenate(
        [bnd, jnp.full((BND_PAD - NRANGE - 1,), E, jnp.int32)])

    x_pad = jnp.pad(x, ((0, NP - N), (0, 0)))
    topo_pad = jnp.pad(topo_hidden, ((0, NP - N), (0, 0)))

    deg = _deg_kernel(col_sp, bnd)

    hs, dinv = _tc_layer0(x_pad, deg, W0, b0.reshape(1, H))
    for W, b in ((W1, b1), (W2, b2), (W3, b3)):
        agg = _agg_kernel(hs, row_s, col_sp, bnd)
        hs = _tc_layer(agg, hs, dinv, W, b.reshape(1, H))
    agg = _agg_kernel(hs, row_s, col_sp, bnd)
    hidden = _tc_final(agg, hs, dinv, topo_pad, Wf, bf.reshape(1, H))

    gmp, gap = _pool_kernel(hidden, batch_index)
    pooled = jnp.concatenate([gmp.reshape(G, H), gap.reshape(G, H)], axis=1)
    out = _tc_head(pooled, Wo, bo.reshape(1, OUT))
    return (out, pooled)
